# initial kernel scaffold (unmeasured)
import jax
import jax.numpy as jnp
from jax import lax
from jax.experimental import pallas as pl
from jax.experimental.pallas import tpu as pltpu

N_DEV = 4
N_EXPERTS = 16
N_LOCAL = N_EXPERTS // N_DEV


def kernel(x, router_W, route_idx, expert_W):
    n, d = x.shape
    h = expert_W.shape[-1]
    chunk = n // N_DEV

    def body(x_ref, rw_ref, idx_ref, ew_ref, out_ref,
             acc_ref, comm_ref, send_sems, recv_sems):
        my = lax.axis_index("i")
        left = lax.rem(my + N_DEV - 1, N_DEV)
        right = lax.rem(my + 1, N_DEV)

        barrier_sem = pltpu.get_barrier_semaphore()
        for nbr in (left, right):
            pl.semaphore_signal(
                barrier_sem, inc=1,
                device_id=(nbr,), device_id_type=pl.DeviceIdType.MESH,
            )
        pl.semaphore_wait(barrier_sem, 2)

        xv = x_ref[...]
        scores = jnp.dot(xv, rw_ref[...],
                         preferred_element_type=jnp.float32)
        m = jnp.max(scores, axis=-1, keepdims=True)
        e = jnp.exp(scores - m)
        probs = e / jnp.sum(e, axis=-1, keepdims=True)
        idx = idx_ref[...]
        eids = lax.broadcasted_iota(jnp.int32, (n, N_EXPERTS), 1)
        onehot = (idx[:, 0:1] == eids) | (idx[:, 1:2] == eids)
        sel = jnp.where(onehot, probs, 0.0)
        gates = sel / jnp.sum(sel, axis=-1, keepdims=True)
        local_gates = lax.dynamic_slice(
            gates, (0, my * N_LOCAL), (n, N_LOCAL))

        acc = jnp.zeros((n, h), jnp.float32)
        for le in range(N_LOCAL):
            xg = (xv * local_gates[:, le][:, None]).astype(jnp.bfloat16)
            w = ew_ref[le, :, :].astype(jnp.bfloat16)
            acc = acc + jnp.dot(xg, w, preferred_element_type=jnp.float32)
        acc_ref[...] = acc

        c0 = lax.rem(my + N_DEV - 1, N_DEV)
        comm_ref[0, :, :] = acc_ref[pl.ds(c0 * chunk, chunk), :]
        for s in range(N_DEV - 1):
            rdma = pltpu.make_async_remote_copy(
                src_ref=comm_ref.at[s],
                dst_ref=comm_ref.at[s + 1],
                send_sem=send_sems.at[s],
                recv_sem=recv_sems.at[s],
                device_id=(right,),
                device_id_type=pl.DeviceIdType.MESH,
            )
            rdma.start()
            rdma.wait()
            c = lax.rem(my + 2 * N_DEV - 2 - s, N_DEV)
            local = acc_ref[pl.ds(c * chunk, chunk), :]
            if s < N_DEV - 2:
                comm_ref[s + 1, :, :] = comm_ref[s + 1, :, :] + local
            else:
                out_ref[...] = comm_ref[s + 1, :, :] + local

    return pl.pallas_call(
        body,
        out_shape=jax.ShapeDtypeStruct((chunk, h), jnp.float32),
        in_specs=[
            pl.BlockSpec(memory_space=pltpu.VMEM),
            pl.BlockSpec(memory_space=pltpu.VMEM),
            pl.BlockSpec(memory_space=pltpu.VMEM),
            pl.BlockSpec(memory_space=pltpu.VMEM),
        ],
        out_specs=pl.BlockSpec(memory_space=pltpu.VMEM),
        scratch_shapes=[
            pltpu.VMEM((n, h), jnp.float32),
            pltpu.VMEM((N_DEV, chunk, h), jnp.float32),
            pltpu.SemaphoreType.DMA((N_DEV - 1,)),
            pltpu.SemaphoreType.DMA((N_DEV - 1,)),
        ],
        compiler_params=pltpu.CompilerParams(collective_id=0),
    )(x, router_W, route_idx, expert_W)


# baseline (device time: 56647 ns/iter reference)
import jax
import jax.numpy as jnp
from jax import lax
from jax.experimental import pallas as pl
from jax.experimental.pallas import tpu as pltpu

N_DEV = 4
N_EXPERTS = 16
N_LOCAL = N_EXPERTS // N_DEV


def kernel(x, router_W, route_idx, expert_W):
    n, d = x.shape
    h = expert_W.shape[-1]
    chunk = n // N_DEV

    def body(x_ref, rw_ref, idx_ref, ew_ref, out_ref,
             acc_ref, comm_ref, send_sems, recv_sems):
        my = lax.axis_index("i")
        left = lax.rem(my + N_DEV - 1, N_DEV)
        right = lax.rem(my + 1, N_DEV)

        barrier_sem = pltpu.get_barrier_semaphore()
        for nbr in (left, right):
            pl.semaphore_signal(
                barrier_sem, inc=1,
                device_id=(nbr,), device_id_type=pl.DeviceIdType.MESH,
            )
        pl.semaphore_wait(barrier_sem, 2)

        xv = x_ref[...]
        scores = jnp.dot(xv, rw_ref[...],
                         preferred_element_type=jnp.float32)
        m = jnp.max(scores, axis=-1, keepdims=True)
        e = jnp.exp(scores - m)
        probs = e / jnp.sum(e, axis=-1, keepdims=True)
        idx = idx_ref[...]
        eids = lax.broadcasted_iota(jnp.int32, (n, N_EXPERTS), 1)
        onehot = (idx[:, 0:1] == eids) | (idx[:, 1:2] == eids)
        sel = jnp.where(onehot, probs, 0.0)
        denom = jnp.sum(sel, axis=-1, keepdims=True)

        acc = jnp.zeros((n, h), jnp.float32)
        for le in range(N_LOCAL):
            eg = my * N_LOCAL + le
            g = jnp.sum(jnp.where(eids == eg, sel, 0.0),
                        axis=-1, keepdims=True) / denom
            xg = (xv * g).astype(jnp.bfloat16)
            w = ew_ref[le, :, :].astype(jnp.bfloat16)
            acc = acc + jnp.dot(xg, w, preferred_element_type=jnp.float32)
        acc_ref[...] = acc

        c0 = lax.rem(my + N_DEV - 1, N_DEV)
        comm_ref[0, :, :] = acc_ref[pl.ds(c0 * chunk, chunk), :]
        for s in range(N_DEV - 1):
            rdma = pltpu.make_async_remote_copy(
                src_ref=comm_ref.at[s],
                dst_ref=comm_ref.at[s + 1],
                send_sem=send_sems.at[s],
                recv_sem=recv_sems.at[s],
                device_id=(right,),
                device_id_type=pl.DeviceIdType.MESH,
            )
            rdma.start()
            rdma.wait()
            c = lax.rem(my + 2 * N_DEV - 2 - s, N_DEV)
            local = acc_ref[pl.ds(c * chunk, chunk), :]
            if s < N_DEV - 2:
                comm_ref[s + 1, :, :] = comm_ref[s + 1, :, :] + local
            else:
                out_ref[...] = comm_ref[s + 1, :, :] + local

    return pl.pallas_call(
        body,
        out_shape=jax.ShapeDtypeStruct((chunk, h), jnp.float32),
        in_specs=[
            pl.BlockSpec(memory_space=pltpu.VMEM),
            pl.BlockSpec(memory_space=pltpu.VMEM),
            pl.BlockSpec(memory_space=pltpu.VMEM),
            pl.BlockSpec(memory_space=pltpu.VMEM),
        ],
        out_specs=pl.BlockSpec(memory_space=pltpu.VMEM),
        scratch_shapes=[
            pltpu.VMEM((n, h), jnp.float32),
            pltpu.VMEM((N_DEV, chunk, h), jnp.float32),
            pltpu.SemaphoreType.DMA((N_DEV - 1,)),
            pltpu.SemaphoreType.DMA((N_DEV - 1,)),
        ],
        compiler_params=pltpu.CompilerParams(collective_id=0),
    )(x, router_W, route_idx, expert_W)


# device time: 37453 ns/iter; 1.5125x vs baseline; 1.5125x over previous
import jax
import jax.numpy as jnp
from jax import lax
from jax.experimental import pallas as pl
from jax.experimental.pallas import tpu as pltpu

N_DEV = 4
N_EXPERTS = 16
N_LOCAL = N_EXPERTS // N_DEV


def kernel(x, router_W, route_idx, expert_W):
    n, d = x.shape
    h = expert_W.shape[-1]
    chunk = n // N_DEV

    def body(x_ref, rw_ref, idx_ref, ew_ref, out_ref,
             lg_ref, wb_ref, comm_ref, send_sems, recv_sems):
        my = lax.axis_index("i")
        left = lax.rem(my + N_DEV - 1, N_DEV)
        right = lax.rem(my + 1, N_DEV)

        barrier_sem = pltpu.get_barrier_semaphore()
        for nbr in (left, right):
            pl.semaphore_signal(
                barrier_sem, inc=1,
                device_id=(nbr,), device_id_type=pl.DeviceIdType.MESH,
            )
        pl.semaphore_wait(barrier_sem, 2)

        xv = x_ref[...]
        scores = jnp.dot(xv, rw_ref[...],
                         preferred_element_type=jnp.float32)
        m = jnp.max(scores, axis=-1, keepdims=True)
        e = jnp.exp(scores - m)
        probs = e / jnp.sum(e, axis=-1, keepdims=True)
        idx = idx_ref[...]
        eids = lax.broadcasted_iota(jnp.int32, (n, N_EXPERTS), 1)
        onehot = (idx[:, 0:1] == eids) | (idx[:, 1:2] == eids)
        sel = jnp.where(onehot, probs, 0.0)
        denom = jnp.sum(sel, axis=-1, keepdims=True)
        cols = []
        for le in range(N_LOCAL):
            eg = my * N_LOCAL + le
            cols.append(jnp.sum(jnp.where(eids == eg, sel, 0.0),
                                axis=-1, keepdims=True) / denom)
        lg_ref[...] = jnp.concatenate(cols, axis=1)
        wb_ref[...] = ew_ref[...].astype(jnp.bfloat16)

        def partial_chunk(c):
            rows = pl.ds(c * chunk, chunk)
            xr = x_ref[rows, :]
            gr = lg_ref[rows, :]
            acc = jnp.zeros((chunk, h), jnp.float32)
            for le in range(N_LOCAL):
                xg = (xr * gr[:, le:le + 1]).astype(jnp.bfloat16)
                acc = acc + jnp.dot(xg, wb_ref[le, :, :],
                                    preferred_element_type=jnp.float32)
            return acc

        comm_ref[0, :, :] = partial_chunk(left).astype(jnp.bfloat16)
        rdmas = []
        for s in range(N_DEV - 1):
            rdma = pltpu.make_async_remote_copy(
                src_ref=comm_ref.at[s],
                dst_ref=comm_ref.at[s + 1],
                send_sem=send_sems.at[s],
                recv_sem=recv_sems.at[s],
                device_id=(right,),
                device_id_type=pl.DeviceIdType.MESH,
            )
            rdma.start()
            rdmas.append(rdma)
            c = lax.rem(my + 2 * N_DEV - 2 - s, N_DEV)
            local = partial_chunk(c)
            rdma.wait_recv()
            if s < N_DEV - 2:
                comm_ref[s + 1, :, :] = (
                    comm_ref[s + 1, :, :] + local.astype(jnp.bfloat16))
            else:
                out_ref[...] = (
                    comm_ref[s + 1, :, :].astype(jnp.float32) + local)
        for rdma in rdmas:
            rdma.wait_send()

    return pl.pallas_call(
        body,
        out_shape=jax.ShapeDtypeStruct((chunk, h), jnp.float32),
        in_specs=[
            pl.BlockSpec(memory_space=pltpu.VMEM),
            pl.BlockSpec(memory_space=pltpu.VMEM),
            pl.BlockSpec(memory_space=pltpu.VMEM),
            pl.BlockSpec(memory_space=pltpu.VMEM),
        ],
        out_specs=pl.BlockSpec(memory_space=pltpu.VMEM),
        scratch_shapes=[
            pltpu.VMEM((n, N_LOCAL), jnp.float32),
            pltpu.VMEM((N_LOCAL, d, h), jnp.bfloat16),
            pltpu.VMEM((N_DEV, chunk, h), jnp.bfloat16),
            pltpu.SemaphoreType.DMA((N_DEV - 1,)),
            pltpu.SemaphoreType.DMA((N_DEV - 1,)),
        ],
        compiler_params=pltpu.CompilerParams(collective_id=0),
    )(x, router_W, route_idx, expert_W)


# device time: 31101 ns/iter; 1.8214x vs baseline; 1.2042x over previous
import jax
import jax.numpy as jnp
from jax import lax
from jax.experimental import pallas as pl
from jax.experimental.pallas import tpu as pltpu

N_DEV = 4
N_EXPERTS = 16
N_LOCAL = N_EXPERTS // N_DEV


def kernel(x, router_W, route_idx, expert_W):
    n, d = x.shape
    h = expert_W.shape[-1]
    chunk = n // N_DEV
    h2 = h // 2

    def body(x_ref, rw_ref, idx_ref, ew_ref, out_ref,
             lg_ref, wb_ref, cw_ref, ccw_ref,
             cw_send, cw_recv, ccw_send, ccw_recv):
        my = lax.axis_index("i")
        left = lax.rem(my + N_DEV - 1, N_DEV)
        right = lax.rem(my + 1, N_DEV)

        barrier_sem = pltpu.get_barrier_semaphore()
        for nbr in (left, right):
            pl.semaphore_signal(
                barrier_sem, inc=1,
                device_id=(nbr,), device_id_type=pl.DeviceIdType.MESH,
            )
        pl.semaphore_wait(barrier_sem, 2)

        xv = x_ref[...]
        scores = jnp.dot(xv, rw_ref[...],
                         preferred_element_type=jnp.float32)
        m = jnp.max(scores, axis=-1, keepdims=True)
        e = jnp.exp(scores - m)
        probs = e / jnp.sum(e, axis=-1, keepdims=True)
        idx = idx_ref[...]
        eids = lax.broadcasted_iota(jnp.int32, (n, N_EXPERTS), 1)
        onehot = (idx[:, 0:1] == eids) | (idx[:, 1:2] == eids)
        sel = jnp.where(onehot, probs, 0.0)
        denom = jnp.sum(sel, axis=-1, keepdims=True)
        cols = []
        for le in range(N_LOCAL):
            eg = my * N_LOCAL + le
            cols.append(jnp.sum(jnp.where(eids == eg, sel, 0.0),
                                axis=-1, keepdims=True) / denom)
        lg_ref[...] = jnp.concatenate(cols, axis=1)
        wb_ref[...] = ew_ref[...].astype(jnp.bfloat16)

        def partial_chunk(c):
            rows = pl.ds(c * chunk, chunk)
            xr = x_ref[rows, :]
            gr = lg_ref[rows, :]
            acc = jnp.zeros((chunk, h), jnp.float32)
            for le in range(N_LOCAL):
                xg = (xr * gr[:, le:le + 1]).astype(jnp.bfloat16)
                acc = acc + jnp.dot(xg, wb_ref[le, :, :],
                                    preferred_element_type=jnp.float32)
            return acc

        def hop(comm, send_sems, recv_sems, s, dst):
            rdma = pltpu.make_async_remote_copy(
                src_ref=comm.at[s],
                dst_ref=comm.at[s + 1],
                send_sem=send_sems.at[s],
                recv_sem=recv_sems.at[s],
                device_id=(dst,),
                device_id_type=pl.DeviceIdType.MESH,
            )
            rdma.start()
            return rdma

        p_left = partial_chunk(left)
        cw_ref[0, :, :] = p_left[:, :h2].astype(jnp.bfloat16)
        cw0 = hop(cw_ref, cw_send, cw_recv, 0, right)

        p_right = partial_chunk(right)
        ccw_ref[0, :, :] = p_right[:, h2:].astype(jnp.bfloat16)
        ccw0 = hop(ccw_ref, ccw_send, ccw_recv, 0, left)

        opp = lax.rem(my + 2, N_DEV)
        p_opp = partial_chunk(opp)
        cw0.wait_recv()
        cw_ref[1, :, :] = cw_ref[1, :, :] + p_opp[:, :h2].astype(jnp.bfloat16)
        cw1 = hop(cw_ref, cw_send, cw_recv, 1, right)
        ccw0.wait_recv()
        ccw_ref[1, :, :] = ccw_ref[1, :, :] + p_opp[:, h2:].astype(jnp.bfloat16)
        ccw1 = hop(ccw_ref, ccw_send, ccw_recv, 1, left)

        cw1.wait_recv()
        cw_ref[2, :, :] = cw_ref[2, :, :] + p_right[:, :h2].astype(jnp.bfloat16)
        cw2 = hop(cw_ref, cw_send, cw_recv, 2, right)
        ccw1.wait_recv()
        ccw_ref[2, :, :] = ccw_ref[2, :, :] + p_left[:, h2:].astype(jnp.bfloat16)
        ccw2 = hop(ccw_ref, ccw_send, ccw_recv, 2, left)

        p_own = partial_chunk(my)
        cw2.wait_recv()
        out_ref[:, :h2] = cw_ref[3, :, :].astype(jnp.float32) + p_own[:, :h2]
        ccw2.wait_recv()
        out_ref[:, h2:] = ccw_ref[3, :, :].astype(jnp.float32) + p_own[:, h2:]

        for rdma in (cw0, cw1, cw2, ccw0, ccw1, ccw2):
            rdma.wait_send()

    return pl.pallas_call(
        body,
        out_shape=jax.ShapeDtypeStruct((chunk, h), jnp.float32),
        in_specs=[
            pl.BlockSpec(memory_space=pltpu.VMEM),
            pl.BlockSpec(memory_space=pltpu.VMEM),
            pl.BlockSpec(memory_space=pltpu.VMEM),
            pl.BlockSpec(memory_space=pltpu.VMEM),
        ],
        out_specs=pl.BlockSpec(memory_space=pltpu.VMEM),
        scratch_shapes=[
            pltpu.VMEM((n, N_LOCAL), jnp.float32),
            pltpu.VMEM((N_LOCAL, d, h), jnp.bfloat16),
            pltpu.VMEM((N_DEV, chunk, h2), jnp.bfloat16),
            pltpu.VMEM((N_DEV, chunk, h2), jnp.bfloat16),
            pltpu.SemaphoreType.DMA((N_DEV - 1,)),
            pltpu.SemaphoreType.DMA((N_DEV - 1,)),
            pltpu.SemaphoreType.DMA((N_DEV - 1,)),
            pltpu.SemaphoreType.DMA((N_DEV - 1,)),
        ],
        compiler_params=pltpu.CompilerParams(collective_id=0),
    )(x, router_W, route_idx, expert_W)
